# baseline (speedup 1.0000x reference)
import functools

import jax
import jax.numpy as jnp
from jax import lax
from jax.experimental import pallas as pl
from jax.experimental.pallas import tpu as pltpu
from jax.experimental.pallas import tpu_sc as plsc

NG = 128
NP = 100
EP = 1600
DN = 128
DE = 16
DG = 693
H = 128
GB = 4


def _ln(x, g, b):
    mu = jnp.mean(x, axis=-1, keepdims=True)
    m2 = jnp.mean(x * x, axis=-1, keepdims=True)
    var = m2 - mu * mu
    return (x - mu) * jax.lax.rsqrt(var + 1e-5) * g + b


def _dot(a, b):
    return jax.lax.dot_general(a.astype(jnp.bfloat16), b.astype(jnp.bfloat16),
                               (((1,), (0,)), ((), ())),
                               preferred_element_type=jnp.float32,
                               precision=jax.lax.Precision.DEFAULT)


def _dotT(a, b):
    return jax.lax.dot_general(a.astype(jnp.bfloat16), b.astype(jnp.bfloat16),
                               (((0,), (0,)), ((), ())),
                               preferred_element_type=jnp.float32,
                               precision=jax.lax.Precision.DEFAULT)


def _fold(p, slices, ones_slices=()):
    W1 = p['W1']
    bn_g = p['bn_g']
    bn_b = p['bn_b']
    const = p['b1'] + bn_b @ W1
    for (s, n) in ones_slices:
        const = const + bn_g[s:s + n] @ W1[s:s + n]
    Ws = [bn_g[s:s + n][:, None] * W1[s:s + n] for (s, n) in slices]
    return Ws, const[None, :]


def _proj(x, W):
    def b(x_ref, w_ref, o_ref):
        o_ref[...] = _dot(x_ref[...], w_ref[...])
    return pl.pallas_call(
        b, grid=(8,),
        in_specs=[pl.BlockSpec((NG * NP // 8, DN), lambda i: (i, 0)),
                  pl.BlockSpec((DN, H), lambda i: (0, 0))],
        out_specs=pl.BlockSpec((NG * NP // 8, H), lambda i: (i, 0)),
        out_shape=jax.ShapeDtypeStruct((NG * NP, H), jnp.float32))(x, W)


def _sc_gather(table, idx):
    info = plsc.get_sparse_core_info()
    NW = info.num_cores * info.num_subcores
    B = idx.shape[0]
    b_w = B // NW
    CH = 128
    n_ch = b_w // CH
    mesh = plsc.VectorSubcoreMesh(core_axis_name="c", subcore_axis_name="s")

    @functools.partial(
        pl.kernel, mesh=mesh,
        out_type=jax.ShapeDtypeStruct((B, H), jnp.float32),
        scratch_types=[pltpu.VMEM((CH,), jnp.int32),
                       pltpu.VMEM((CH, H), jnp.float32),
                       pltpu.SemaphoreType.DMA])
    def k(tab_hbm, idx_hbm, out_hbm, idx_v, rows_v, sem):
        wid = lax.axis_index("s") * info.num_cores + lax.axis_index("c")

        def step(j, c):
            off = wid * b_w + j * CH
            pltpu.sync_copy(idx_hbm.at[pl.ds(off, CH)], idx_v)
            pltpu.async_copy(tab_hbm.at[idx_v], rows_v, sem).wait()
            pltpu.sync_copy(rows_v, out_hbm.at[pl.ds(off, CH)])
            return c

        lax.fori_loop(0, n_ch, step, 0)

    return k(table, idx)


def kernel(nodes, edges, senders, receivers, node_gid, edge_gid, params):
    f32 = jnp.float32

    p1e, p1n, p1g = params['dec1']['edge'], params['dec1']['node'], params['dec1']['glob']
    p2e, p2n, p2g = params['dec2']['edge'], params['dec2']['node'], params['dec2']['glob']

    (We1e, We1r, We1s), c_e1 = _fold(
        p1e, [(0, DE), (DE, DN), (DE + DN, DN)], [(DE + 2 * DN, DG)])
    (Wn1a, Wn1n), c_n1 = _fold(
        p1n, [(0, H), (H, DN)], [(H + DN, DG)])
    (Wg1e, Wg1n), c_g1 = _fold(
        p1g, [(0, H), (H, H)], [(2 * H, DG)])
    (We2e, We2r, We2s, We2g), c_e2 = _fold(
        p2e, [(0, H), (H, H), (2 * H, H), (3 * H, DG)])
    (Wn2a, Wn2n, Wn2g), c_n2 = _fold(
        p2n, [(0, H), (H, H), (2 * H, DG)])
    (Wg2e, Wg2n, Wg2g), c_g2 = _fold(
        p2g, [(0, H), (H, H), (2 * H, DG)])

    def tail(p):
        return [p['W2'], p['b2'][None, :], p['ln_g'][None, :], p['ln_b'][None, :]]

    weights = (
        [We1e, We1r, We1s, c_e1] + tail(p1e) +
        [Wn1a, Wn1n, c_n1] + tail(p1n) +
        [Wg1e, Wg1n, c_g1] + tail(p1g) +
        [We2e, We2r, We2s, We2g, c_e2] + tail(p2e) +
        [Wn2a, Wn2n, Wn2g, c_n2] + tail(p2n) +
        [Wg2e, Wg2n, Wg2g, c_g2] + tail(p2g)
    )

    base = edge_gid * NP
    r_row = (receivers - base).astype(jnp.int32).reshape(NG, 1, EP)
    s_row = (senders - base).astype(jnp.int32).reshape(NG, 1, EP)

    def body(nodes_ref, edges_ref, rrow_ref, srow_ref, gr_ref,
             We1e_ref, We1r_ref, We1s_ref, ce1_ref, W2e1_ref, be1_ref, ge1_ref, le1_ref,
             Wn1a_ref, Wn1n_ref, cn1_ref, W2n1_ref, bn1_ref, gn1_ref, ln1_ref,
             Wg1e_ref, Wg1n_ref, cg1_ref, W2g1_ref, bg1_ref, gg1_ref, lg1_ref,
             We2e_ref, We2r_ref, We2s_ref, We2g_ref, ce2_ref, W2e2_ref, be2_ref, ge2_ref, le2_ref,
             Wn2a_ref, Wn2n_ref, Wn2g_ref, cn2_ref, W2n2_ref, bn2_ref, gn2_ref, ln2_ref,
             Wg2e_ref, Wg2n_ref, Wg2g_ref, cg2_ref, W2g2_ref, bg2_ref, gg2_ref, lg2_ref,
             n2_ref, e2_ref, g2_ref):
      for i in range(GB):
        nod = nodes_ref[pl.ds(i * NP, NP), :]
        edgT = edges_ref[:, pl.ds(i * EP, EP)]
        rrow = rrow_ref[i]
        srow = srow_ref[i]

        sub_n = jax.lax.broadcasted_iota(jnp.int32, (NP, EP), 0)
        ohT_r = (sub_n == rrow).astype(f32)
        ohT_s = (sub_n == srow).astype(f32)

        gr = gr_ref[pl.ds(i * EP, EP), :]
        Ps = _dot(nod, We1s_ref[...])
        h = _dotT(edgT, We1e_ref[...]) + gr + _dotT(ohT_s, Ps) + ce1_ref[...]
        h = jnp.maximum(h, 0.0)
        h = jnp.maximum(_dot(h, W2e1_ref[...]) + be1_ref[...], 0.0)
        e1 = _ln(h, ge1_ref[...], le1_ref[...])

        agg_e2n = _dot(ohT_r, e1)
        agg_e2g = jnp.sum(e1, axis=0, keepdims=True)

        h = _dot(agg_e2n, Wn1a_ref[...]) + _dot(nod, Wn1n_ref[...]) + cn1_ref[...]
        h = jnp.maximum(h, 0.0)
        h = jnp.maximum(_dot(h, W2n1_ref[...]) + bn1_ref[...], 0.0)
        n1 = _ln(h, gn1_ref[...], ln1_ref[...])
        agg_n2g = jnp.sum(n1, axis=0, keepdims=True)

        h = _dot(agg_e2g, Wg1e_ref[...]) + _dot(agg_n2g, Wg1n_ref[...]) + cg1_ref[...]
        h = jnp.maximum(h, 0.0)
        h = jnp.maximum(_dot(h, W2g1_ref[...]) + bg1_ref[...], 0.0)
        g1 = _ln(h, gg1_ref[...], lg1_ref[...])

        Pr2 = _dot(n1, We2r_ref[...])
        Ps2 = _dot(n1, We2s_ref[...])
        gproj = _dot(g1, We2g_ref[...])
        h = (_dot(e1, We2e_ref[...]) + _dotT(ohT_r, Pr2) + _dotT(ohT_s, Ps2)
             + gproj + ce2_ref[...])
        h = jnp.maximum(h, 0.0)
        h = jnp.maximum(_dot(h, W2e2_ref[...]) + be2_ref[...], 0.0)
        e2 = _ln(h, ge2_ref[...], le2_ref[...])

        agg2_e2n = _dot(ohT_r, e2)
        agg2_e2g = jnp.sum(e2, axis=0, keepdims=True)

        gprojn = _dot(g1, Wn2g_ref[...])
        h = (_dot(agg2_e2n, Wn2a_ref[...]) + _dot(n1, Wn2n_ref[...])
             + gprojn + cn2_ref[...])
        h = jnp.maximum(h, 0.0)
        h = jnp.maximum(_dot(h, W2n2_ref[...]) + bn2_ref[...], 0.0)
        n2 = _ln(h, gn2_ref[...], ln2_ref[...])
        agg2_n2g = jnp.sum(n2, axis=0, keepdims=True)

        h = (_dot(agg2_e2g, Wg2e_ref[...]) + _dot(agg2_n2g, Wg2n_ref[...])
             + _dot(g1, Wg2g_ref[...]) + cg2_ref[...])
        h = jnp.maximum(h, 0.0)
        h = jnp.maximum(_dot(h, W2g2_ref[...]) + bg2_ref[...], 0.0)
        g2 = _ln(h, gg2_ref[...], lg2_ref[...])

        lane_h = jax.lax.broadcasted_iota(jnp.int32, (EP, H), 1)
        e2_out = jnp.where(lane_h == 0, jax.nn.sigmoid(e2), e2)

        n2_ref[pl.ds(i * NP, NP), :] = n2
        e2_ref[pl.ds(i * EP, EP), :] = e2_out
        g2_ref[i] = g2

    def rspec(rows, cols):
        return pl.BlockSpec((GB * rows, cols), lambda g: (g, 0))

    def wspec(w):
        return pl.BlockSpec(w.shape, lambda g: (0, 0))

    ispec = pl.BlockSpec((GB, 1, EP), lambda g: (g, 0, 0))
    espec = pl.BlockSpec((DE, GB * EP), lambda g: (0, g))
    in_specs = ([rspec(NP, DN), espec, ispec, ispec, rspec(EP, H)] +
                [wspec(w) for w in weights])
    out_specs = [rspec(NP, H), rspec(EP, H),
                 pl.BlockSpec((GB, 1, DG), lambda g: (g, 0, 0))]
    out_shapes = [jax.ShapeDtypeStruct((NG * NP, H), f32),
                  jax.ShapeDtypeStruct((NG * EP, H), f32),
                  jax.ShapeDtypeStruct((NG, 1, DG), f32)]

    Pr_full = _proj(nodes, We1r)
    gath_r = _sc_gather(Pr_full, receivers.astype(jnp.int32))

    n2, e2, g2p = pl.pallas_call(
        body,
        grid=(NG // GB,),
        in_specs=in_specs,
        out_specs=out_specs,
        out_shape=out_shapes,
    )(nodes, edges.T, r_row, s_row, gath_r, *weights)

    g2 = g2p.reshape(NG, DG)
    mu = jnp.ones((NG, DG), f32)
    logvar = jnp.ones((NG, DG), f32)
    return (n2, e2, g2, mu, logvar)


# speedup vs baseline: 1.1646x; 1.1646x over previous
import jax
import jax.numpy as jnp
from jax.experimental import pallas as pl

NG = 128
NP = 100
EP = 1600
DN = 128
DE = 16
DG = 693
H = 128
GB = 8


def _ln(x, g, b):
    mu = jnp.mean(x, axis=-1, keepdims=True)
    m2 = jnp.mean(x * x, axis=-1, keepdims=True)
    var = m2 - mu * mu
    return (x - mu) * jax.lax.rsqrt(var + 1e-5) * g + b


def _dot(a, b):
    return jax.lax.dot_general(a.astype(jnp.bfloat16), b.astype(jnp.bfloat16),
                               (((1,), (0,)), ((), ())),
                               preferred_element_type=jnp.float32,
                               precision=jax.lax.Precision.DEFAULT)


def _dotT(a, b):
    return jax.lax.dot_general(a.astype(jnp.bfloat16), b.astype(jnp.bfloat16),
                               (((0,), (0,)), ((), ())),
                               preferred_element_type=jnp.float32,
                               precision=jax.lax.Precision.DEFAULT)


def _fold(p, slices, ones_slices=()):
    W1 = p['W1']
    bn_g = p['bn_g']
    bn_b = p['bn_b']
    const = p['b1'] + bn_b @ W1
    for (s, n) in ones_slices:
        const = const + bn_g[s:s + n] @ W1[s:s + n]
    Ws = [bn_g[s:s + n][:, None] * W1[s:s + n] for (s, n) in slices]
    return Ws, const[None, :]


def kernel(nodes, edges, senders, receivers, node_gid, edge_gid, params):
    f32 = jnp.float32

    p1e, p1n, p1g = params['dec1']['edge'], params['dec1']['node'], params['dec1']['glob']
    p2e, p2n, p2g = params['dec2']['edge'], params['dec2']['node'], params['dec2']['glob']

    (We1e, We1r, We1s), c_e1 = _fold(
        p1e, [(0, DE), (DE, DN), (DE + DN, DN)], [(DE + 2 * DN, DG)])
    (Wn1a, Wn1n), c_n1 = _fold(
        p1n, [(0, H), (H, DN)], [(H + DN, DG)])
    (Wg1e, Wg1n), c_g1 = _fold(
        p1g, [(0, H), (H, H)], [(2 * H, DG)])
    (We2e, We2r, We2s, We2g), c_e2 = _fold(
        p2e, [(0, H), (H, H), (2 * H, H), (3 * H, DG)])
    (Wn2a, Wn2n, Wn2g), c_n2 = _fold(
        p2n, [(0, H), (H, H), (2 * H, DG)])
    (Wg2e, Wg2n, Wg2g), c_g2 = _fold(
        p2g, [(0, H), (H, H), (2 * H, DG)])

    def tail(p):
        return [p['W2'], p['b2'][None, :], p['ln_g'][None, :], p['ln_b'][None, :]]

    weights = (
        [We1e, We1r, We1s, c_e1] + tail(p1e) +
        [Wn1a, Wn1n, c_n1] + tail(p1n) +
        [Wg1e, Wg1n, c_g1] + tail(p1g) +
        [We2e, We2r, We2s, We2g, c_e2] + tail(p2e) +
        [Wn2a, Wn2n, Wn2g, c_n2] + tail(p2n) +
        [Wg2e, Wg2n, Wg2g, c_g2] + tail(p2g)
    )

    base = edge_gid * NP
    r_row = (receivers - base).astype(jnp.int32).reshape(NG, 1, EP)
    s_row = (senders - base).astype(jnp.int32).reshape(NG, 1, EP)

    def body(nodes_ref, edges_ref, rrow_ref, srow_ref,
             We1e_ref, We1r_ref, We1s_ref, ce1_ref, W2e1_ref, be1_ref, ge1_ref, le1_ref,
             Wn1a_ref, Wn1n_ref, cn1_ref, W2n1_ref, bn1_ref, gn1_ref, ln1_ref,
             Wg1e_ref, Wg1n_ref, cg1_ref, W2g1_ref, bg1_ref, gg1_ref, lg1_ref,
             We2e_ref, We2r_ref, We2s_ref, We2g_ref, ce2_ref, W2e2_ref, be2_ref, ge2_ref, le2_ref,
             Wn2a_ref, Wn2n_ref, Wn2g_ref, cn2_ref, W2n2_ref, bn2_ref, gn2_ref, ln2_ref,
             Wg2e_ref, Wg2n_ref, Wg2g_ref, cg2_ref, W2g2_ref, bg2_ref, gg2_ref, lg2_ref,
             n2_ref, e2_ref, g2_ref):
      for i in range(GB):
        nod = nodes_ref[pl.ds(i * NP, NP), :]
        edgT = edges_ref[:, pl.ds(i * EP, EP)]
        rrow = rrow_ref[i]
        srow = srow_ref[i]

        sub_n = jax.lax.broadcasted_iota(jnp.int32, (NP, EP), 0)
        ohT_r = (sub_n == rrow).astype(f32)
        ohT_s = (sub_n == srow).astype(f32)

        Pr = _dot(nod, We1r_ref[...])
        Ps = _dot(nod, We1s_ref[...])
        h = _dotT(edgT, We1e_ref[...]) + _dotT(ohT_r, Pr) + _dotT(ohT_s, Ps) + ce1_ref[...]
        h = jnp.maximum(h, 0.0)
        h = jnp.maximum(_dot(h, W2e1_ref[...]) + be1_ref[...], 0.0)
        e1 = _ln(h, ge1_ref[...], le1_ref[...])

        agg_e2n = _dot(ohT_r, e1)
        agg_e2g = jnp.sum(e1, axis=0, keepdims=True)

        h = _dot(agg_e2n, Wn1a_ref[...]) + _dot(nod, Wn1n_ref[...]) + cn1_ref[...]
        h = jnp.maximum(h, 0.0)
        h = jnp.maximum(_dot(h, W2n1_ref[...]) + bn1_ref[...], 0.0)
        n1 = _ln(h, gn1_ref[...], ln1_ref[...])
        agg_n2g = jnp.sum(n1, axis=0, keepdims=True)

        h = _dot(agg_e2g, Wg1e_ref[...]) + _dot(agg_n2g, Wg1n_ref[...]) + cg1_ref[...]
        h = jnp.maximum(h, 0.0)
        h = jnp.maximum(_dot(h, W2g1_ref[...]) + bg1_ref[...], 0.0)
        g1 = _ln(h, gg1_ref[...], lg1_ref[...])

        Pr2 = _dot(n1, We2r_ref[...])
        Ps2 = _dot(n1, We2s_ref[...])
        gproj = _dot(g1, We2g_ref[...])
        h = (_dot(e1, We2e_ref[...]) + _dotT(ohT_r, Pr2) + _dotT(ohT_s, Ps2)
             + gproj + ce2_ref[...])
        h = jnp.maximum(h, 0.0)
        h = jnp.maximum(_dot(h, W2e2_ref[...]) + be2_ref[...], 0.0)
        e2 = _ln(h, ge2_ref[...], le2_ref[...])

        agg2_e2n = _dot(ohT_r, e2)
        agg2_e2g = jnp.sum(e2, axis=0, keepdims=True)

        gprojn = _dot(g1, Wn2g_ref[...])
        h = (_dot(agg2_e2n, Wn2a_ref[...]) + _dot(n1, Wn2n_ref[...])
             + gprojn + cn2_ref[...])
        h = jnp.maximum(h, 0.0)
        h = jnp.maximum(_dot(h, W2n2_ref[...]) + bn2_ref[...], 0.0)
        n2 = _ln(h, gn2_ref[...], ln2_ref[...])
        agg2_n2g = jnp.sum(n2, axis=0, keepdims=True)

        h = (_dot(agg2_e2g, Wg2e_ref[...]) + _dot(agg2_n2g, Wg2n_ref[...])
             + _dot(g1, Wg2g_ref[...]) + cg2_ref[...])
        h = jnp.maximum(h, 0.0)
        h = jnp.maximum(_dot(h, W2g2_ref[...]) + bg2_ref[...], 0.0)
        g2 = _ln(h, gg2_ref[...], lg2_ref[...])

        lane_h = jax.lax.broadcasted_iota(jnp.int32, (EP, H), 1)
        e2_out = jnp.where(lane_h == 0, jax.nn.sigmoid(e2), e2)

        n2_ref[pl.ds(i * NP, NP), :] = n2
        e2_ref[pl.ds(i * EP, EP), :] = e2_out
        g2_ref[i] = g2

    def rspec(rows, cols):
        return pl.BlockSpec((GB * rows, cols), lambda g: (g, 0))

    def wspec(w):
        return pl.BlockSpec(w.shape, lambda g: (0, 0))

    ispec = pl.BlockSpec((GB, 1, EP), lambda g: (g, 0, 0))
    espec = pl.BlockSpec((DE, GB * EP), lambda g: (0, g))
    in_specs = ([rspec(NP, DN), espec, ispec, ispec] +
                [wspec(w) for w in weights])
    out_specs = [rspec(NP, H), rspec(EP, H),
                 pl.BlockSpec((GB, 1, DG), lambda g: (g, 0, 0))]
    out_shapes = [jax.ShapeDtypeStruct((NG * NP, H), f32),
                  jax.ShapeDtypeStruct((NG * EP, H), f32),
                  jax.ShapeDtypeStruct((NG, 1, DG), f32)]

    n2, e2, g2p = pl.pallas_call(
        body,
        grid=(NG // GB,),
        in_specs=in_specs,
        out_specs=out_specs,
        out_shape=out_shapes,
    )(nodes, edges.T, r_row, s_row, *weights)

    g2 = g2p.reshape(NG, DG)
    mu = jnp.ones((NG, DG), f32)
    logvar = jnp.ones((NG, DG), f32)
    return (n2, e2, g2, mu, logvar)
